# baseline (speedup 1.0000x reference)
import jax
import jax.numpy as jnp
from jax.experimental import pallas as pl
from jax.experimental.pallas import tpu as pltpu


def _make_kernel(NP, NW, RB):
    def _fused(x1_ref, x2_ref, wr_ref, br_ref, we_ref, be_ref, out_ref,
               acc_ref, buf_ref, sems):
        i = pl.program_id(0)

        @pl.when(i < NP)
        def _():
            parts = []
            for x_ref in (x1_ref, x2_ref):
                xb = x_ref[...]
                logits = jax.lax.dot_general(
                    xb, wr_ref[...], (((1,), (1,)), ((), ())),
                    preferred_element_type=jnp.float32)
                logits = logits + br_ref[...]
                m = jnp.max(logits, axis=1, keepdims=True)
                denom = jnp.sum(jnp.exp(logits - m), axis=1, keepdims=True)
                G = 1.0 / denom
                u = jax.lax.dot_general(
                    G, xb, (((0,), (0,)), ((), ())),
                    preferred_element_type=jnp.float32)
                gsum = jnp.sum(G, axis=0, keepdims=True)
                parts.append(jnp.concatenate(
                    [u, jnp.broadcast_to(gsum, u.shape)], axis=0))
            part = parts[0] + parts[1]

            @pl.when(i == 0)
            def _():
                acc_ref[...] = part

            @pl.when(i != 0)
            def _():
                acc_ref[...] += part

        @pl.when(i == NP)
        def _():
            u = acc_ref[0:1, :]
            g = acc_ref[1, 0]
            v = jax.lax.dot_general(
                u, we_ref[...], (((1,), (1,)), ((), ())),
                preferred_element_type=jnp.float32)
            v = v + g * be_ref[...]
            buf_ref[...] = jnp.broadcast_to(v, buf_ref.shape)
            copies = []
            for r in range(NW):
                c = pltpu.make_async_copy(
                    buf_ref, out_ref.at[pl.ds(r * RB, RB), :], sems.at[r])
                c.start()
                copies.append(c)
            for c in copies:
                c.wait()

    return _fused


def kernel(x, W_r, b_r, W_e, b_e):
    b, s, h = x.shape
    bs = b * s
    e = W_r.shape[0]
    xf = x.reshape(bs, h)
    br2 = b_r.reshape(1, e)
    be2 = b_e.reshape(1, h)

    TB = 1024
    RB = 512
    NP = bs // (2 * TB)
    NW = bs // RB

    yflat = pl.pallas_call(
        _make_kernel(NP, NW, RB),
        grid=(NP + 1,),
        in_specs=[
            pl.BlockSpec((TB, h), lambda i: (jnp.minimum(i, NP - 1), 0)),
            pl.BlockSpec((TB, h), lambda i: (jnp.minimum(i, NP - 1) + NP, 0)),
            pl.BlockSpec((e, h), lambda i: (0, 0)),
            pl.BlockSpec((1, e), lambda i: (0, 0)),
            pl.BlockSpec((h, h), lambda i: (0, 0)),
            pl.BlockSpec((1, h), lambda i: (0, 0)),
        ],
        out_specs=pl.BlockSpec(memory_space=pl.ANY),
        out_shape=jax.ShapeDtypeStruct((bs, h), jnp.float32),
        scratch_shapes=[
            pltpu.VMEM((2, h), jnp.float32),
            pltpu.VMEM((RB, h), jnp.float32),
            pltpu.SemaphoreType.DMA((bs // RB,)),
        ],
        compiler_params=pltpu.CompilerParams(
            dimension_semantics=("arbitrary",)),
    )(xf, xf, W_r, br2, W_e, be2)

    return yflat.reshape(b, s, h)


# speedup vs baseline: 1.1727x; 1.1727x over previous
import jax
import jax.numpy as jnp
from jax.experimental import pallas as pl
from jax.experimental.pallas import tpu as pltpu


def _make_kernel(NR, NW, RB, CH):
    def _fused(x_ref, wr_ref, br_ref, we_ref, be_ref, out_ref,
               acc_ref, web0_ref, web1_ref, sb0_ref, sb1_ref,
               wesems, wsem0, wsem1):
        i = pl.program_id(0)

        def we_copy(c, webuf):
            return pltpu.make_async_copy(
                we_ref.at[pl.ds(c * CH, CH), :], webuf, wesems.at[c])

        @pl.when(i < NR)
        def _():
            xb = x_ref[...]
            logits = jax.lax.dot_general(
                xb, wr_ref[...], (((1,), (1,)), ((), ())),
                preferred_element_type=jnp.float32)
            logits = logits + br_ref[...]
            m = jnp.max(logits, axis=1, keepdims=True)
            denom = jnp.sum(jnp.exp(logits - m), axis=1, keepdims=True)
            G = 1.0 / denom
            u = jax.lax.dot_general(
                G, xb, (((0,), (0,)), ((), ())),
                preferred_element_type=jnp.float32)
            gsum = jnp.sum(G, axis=0, keepdims=True)
            part = jnp.concatenate(
                [u, jnp.broadcast_to(gsum, u.shape)], axis=0)

            @pl.when(i == 0)
            def _():
                acc_ref[...] = part

            @pl.when(i != 0)
            def _():
                acc_ref[...] += part

        @pl.when(i == NR - 2)
        def _():
            we_copy(0, web0_ref).start()

        def chunk(c, webuf, sbuf, wsem):
            we_copy(c, webuf).wait()
            u = acc_ref[0:1, :]
            g = acc_ref[1, 0]
            v = jax.lax.dot_general(
                u, webuf[...], (((1,), (1,)), ((), ())),
                preferred_element_type=jnp.float32)
            v = v + g * be_ref[0:1, pl.ds(c * CH, CH)]
            sbuf[...] = jnp.broadcast_to(v, sbuf.shape)
            for r in range(NW):
                pltpu.make_async_copy(
                    sbuf,
                    out_ref.at[pl.ds(r * RB, RB), pl.ds(c * CH, CH)],
                    wsem.at[r]).start()

        @pl.when(i == NR)
        def _():
            we_copy(1, web1_ref).start()
            chunk(0, web0_ref, sb0_ref, wsem0)

        @pl.when(i == NR + 1)
        def _():
            chunk(1, web1_ref, sb1_ref, wsem1)
            for r in range(NW):
                pltpu.make_async_copy(
                    sb0_ref,
                    out_ref.at[pl.ds(r * RB, RB), pl.ds(0, CH)],
                    wsem0.at[r]).wait()
                pltpu.make_async_copy(
                    sb1_ref,
                    out_ref.at[pl.ds(r * RB, RB), pl.ds(CH, CH)],
                    wsem1.at[r]).wait()

    return _fused


def kernel(x, W_r, b_r, W_e, b_e):
    b, s, h = x.shape
    bs = b * s
    e = W_r.shape[0]
    xf = x.reshape(bs, h)
    br2 = b_r.reshape(1, e)
    be2 = b_e.reshape(1, h)

    TB = 1024
    RB = 512
    CH = h // 2
    NR = bs // TB
    NW = bs // RB

    yflat = pl.pallas_call(
        _make_kernel(NR, NW, RB, CH),
        grid=(NR + 2,),
        in_specs=[
            pl.BlockSpec((TB, h), lambda i: (jnp.minimum(i, NR - 1), 0)),
            pl.BlockSpec((e, h), lambda i: (0, 0)),
            pl.BlockSpec((1, e), lambda i: (0, 0)),
            pl.BlockSpec(memory_space=pl.ANY),
            pl.BlockSpec((1, h), lambda i: (0, 0)),
        ],
        out_specs=pl.BlockSpec(memory_space=pl.ANY),
        out_shape=jax.ShapeDtypeStruct((bs, h), jnp.float32),
        scratch_shapes=[
            pltpu.VMEM((2, h), jnp.float32),
            pltpu.VMEM((CH, h), jnp.float32),
            pltpu.VMEM((CH, h), jnp.float32),
            pltpu.VMEM((RB, CH), jnp.float32),
            pltpu.VMEM((RB, CH), jnp.float32),
            pltpu.SemaphoreType.DMA((2,)),
            pltpu.SemaphoreType.DMA((bs // RB,)),
            pltpu.SemaphoreType.DMA((bs // RB,)),
        ],
        compiler_params=pltpu.CompilerParams(
            dimension_semantics=("arbitrary",)),
    )(xf, W_r, br2, W_e, be2)

    return yflat.reshape(b, s, h)


# speedup vs baseline: 1.1728x; 1.0001x over previous
import jax
import jax.numpy as jnp
from jax.experimental import pallas as pl
from jax.experimental.pallas import tpu as pltpu


def _make_kernel(NR, NW, RB, CH):
    def _fused(x_ref, wr_ref, br_ref, we_ref, be_ref, out_ref,
               acc_ref, web0_ref, web1_ref, sb0_ref, sb1_ref,
               wesems, wsem0, wsem1):
        i = pl.program_id(0)

        def we_copy(c, webuf):
            return pltpu.make_async_copy(
                we_ref.at[pl.ds(c * CH, CH), :], webuf, wesems.at[c])

        @pl.when(i < NR)
        def _():
            xb = x_ref[...]
            logits = jax.lax.dot_general(
                xb, wr_ref[...], (((1,), (1,)), ((), ())),
                preferred_element_type=jnp.float32)
            logits = logits + br_ref[...]
            m = jnp.max(logits, axis=1, keepdims=True)
            denom = jnp.sum(jnp.exp(logits - m), axis=1, keepdims=True)
            G = 1.0 / denom
            u = jax.lax.dot_general(
                G, xb, (((0,), (0,)), ((), ())),
                preferred_element_type=jnp.float32)
            gsum = jnp.sum(G, axis=0, keepdims=True)
            part = jnp.concatenate(
                [u, jnp.broadcast_to(gsum, u.shape)], axis=0)

            @pl.when(i == 0)
            def _():
                acc_ref[...] = part

            @pl.when(i != 0)
            def _():
                acc_ref[...] += part

        @pl.when(i == NR - 2)
        def _():
            we_copy(0, web0_ref).start()

        def chunk(c, webuf, sbuf, wsem):
            we_copy(c, webuf).wait()
            u = acc_ref[0:1, :]
            g = acc_ref[1, 0]
            v = jax.lax.dot_general(
                u, webuf[...], (((1,), (1,)), ((), ())),
                preferred_element_type=jnp.float32)
            v = v + g * be_ref[0:1, pl.ds(c * CH, CH)]
            sbuf[...] = jnp.broadcast_to(v, sbuf.shape)
            for r in range(NW):
                pltpu.make_async_copy(
                    sbuf,
                    out_ref.at[pl.ds(r * RB, RB), pl.ds(c * CH, CH)],
                    wsem.at[r]).start()

        @pl.when(i == NR)
        def _():
            we_copy(1, web1_ref).start()
            chunk(0, web0_ref, sb0_ref, wsem0)

        @pl.when(i == NR + 1)
        def _():
            chunk(1, web1_ref, sb1_ref, wsem1)
            for r in range(NW):
                pltpu.make_async_copy(
                    sb0_ref,
                    out_ref.at[pl.ds(r * RB, RB), pl.ds(0, CH)],
                    wsem0.at[r]).wait()
                pltpu.make_async_copy(
                    sb1_ref,
                    out_ref.at[pl.ds(r * RB, RB), pl.ds(CH, CH)],
                    wsem1.at[r]).wait()

    return _fused


def kernel(x, W_r, b_r, W_e, b_e):
    b, s, h = x.shape
    bs = b * s
    e = W_r.shape[0]
    xf = x.reshape(bs, h)
    br2 = b_r.reshape(1, e)
    be2 = b_e.reshape(1, h)

    TB = 2048
    RB = 512
    CH = h // 2
    NR = bs // TB
    NW = bs // RB

    yflat = pl.pallas_call(
        _make_kernel(NR, NW, RB, CH),
        grid=(NR + 2,),
        in_specs=[
            pl.BlockSpec((TB, h), lambda i: (jnp.minimum(i, NR - 1), 0)),
            pl.BlockSpec((e, h), lambda i: (0, 0)),
            pl.BlockSpec((1, e), lambda i: (0, 0)),
            pl.BlockSpec(memory_space=pl.ANY),
            pl.BlockSpec((1, h), lambda i: (0, 0)),
        ],
        out_specs=pl.BlockSpec(memory_space=pl.ANY),
        out_shape=jax.ShapeDtypeStruct((bs, h), jnp.float32),
        scratch_shapes=[
            pltpu.VMEM((2, h), jnp.float32),
            pltpu.VMEM((CH, h), jnp.float32),
            pltpu.VMEM((CH, h), jnp.float32),
            pltpu.VMEM((RB, CH), jnp.float32),
            pltpu.VMEM((RB, CH), jnp.float32),
            pltpu.SemaphoreType.DMA((2,)),
            pltpu.SemaphoreType.DMA((bs // RB,)),
            pltpu.SemaphoreType.DMA((bs // RB,)),
        ],
        compiler_params=pltpu.CompilerParams(
            dimension_semantics=("arbitrary",)),
    )(xf, W_r, br2, W_e, be2)

    return yflat.reshape(b, s, h)
